# Initial kernel scaffold; baseline (speedup 1.0000x reference)
import jax
import jax.numpy as jnp
from jax.experimental import pallas as pl


def kernel(xyz):
    raise NotImplementedError("write your pallas kernel here")


# speedup vs baseline: 19.8005x; 19.8005x over previous
import functools

import jax
import jax.numpy as jnp
from jax import lax
from jax.experimental import pallas as pl
from jax.experimental.pallas import tpu as pltpu
from jax.experimental.pallas import tpu_sc as plsc

B = 32
N = 8192
G = 512
M = 32
GC = 64
NPROG = B * (G // GC)


def _fps_body(x_ref, y_ref, z_ref, idx_ref, cx_ref, cy_ref, cz_ref):
    x = x_ref[:]
    y = y_ref[:]
    z = z_ref[:]
    col = lax.broadcasted_iota(jnp.int32, (B, N), 1)
    gcol = lax.broadcasted_iota(jnp.int32, (B, G), 1)

    lx = x[:, 0:1]
    ly = y[:, 0:1]
    lz = z[:, 0:1]
    idx_acc = jnp.zeros((B, G), jnp.int32)
    cx_acc = jnp.where(gcol == 0, lx, 0.0)
    cy_acc = jnp.where(gcol == 0, ly, 0.0)
    cz_acc = jnp.where(gcol == 0, lz, 0.0)
    dists = jnp.full((B, N), 1e10, jnp.float32)

    def body(i, carry):
        dists, lx, ly, lz, idx_acc, cx_acc, cy_acc, cz_acc = carry
        dx = x - lx
        dy = y - ly
        dz = z - lz
        d = (dx * dx + dy * dy) + dz * dz
        dists = jnp.minimum(dists, d)
        m = jnp.max(dists, axis=1, keepdims=True)
        nxt = jnp.min(jnp.where(dists == m, col, N), axis=1, keepdims=True)
        onehot = col == nxt
        lx = jnp.sum(jnp.where(onehot, x, 0.0), axis=1, keepdims=True)
        ly = jnp.sum(jnp.where(onehot, y, 0.0), axis=1, keepdims=True)
        lz = jnp.sum(jnp.where(onehot, z, 0.0), axis=1, keepdims=True)
        sel = gcol == i
        idx_acc = jnp.where(sel, nxt, idx_acc)
        cx_acc = jnp.where(sel, lx, cx_acc)
        cy_acc = jnp.where(sel, ly, cy_acc)
        cz_acc = jnp.where(sel, lz, cz_acc)
        return (dists, lx, ly, lz, idx_acc, cx_acc, cy_acc, cz_acc)

    carry = lax.fori_loop(
        1, G, body, (dists, lx, ly, lz, idx_acc, cx_acc, cy_acc, cz_acc))
    idx_ref[:] = carry[4]
    cx_ref[:] = carry[5]
    cy_ref[:] = carry[6]
    cz_ref[:] = carry[7]


def _fps_call(x, y, z):
    out_shape = (
        jax.ShapeDtypeStruct((B, G), jnp.int32),
        jax.ShapeDtypeStruct((B, G), jnp.float32),
        jax.ShapeDtypeStruct((B, G), jnp.float32),
        jax.ShapeDtypeStruct((B, G), jnp.float32),
    )
    return pl.pallas_call(_fps_body, out_shape=out_shape)(x, y, z)


NL = 128
NCH = N // NL
NR = 8
_INF = float("inf")
_IBIG = 2**30


def _knn_body(x_ref, y_ref, z_ref, c_ref, out_ref):
    bi = pl.program_id(0)
    xr = x_ref[pl.ds(bi, 1), :]
    yr = y_ref[pl.ds(bi, 1), :]
    zr = z_ref[pl.ds(bi, 1), :]
    x2 = (xr * xr + yr * yr) + zr * zr
    pb = jnp.concatenate(
        [xr, yr, zr, jnp.zeros((5, N), jnp.float32)], axis=0)

    def chunk(gc, _):
        cb = c_ref[0, pl.ds(gc * GC, GC), :]
        cxc = cb[:, 0:1]
        cyc = cb[:, 1:2]
        czc = cb[:, 2:3]
        c2 = (cxc * cxc + cyc * cyc) + czc * czc
        ca = jnp.concatenate(
            [cxc, cyc, czc, jnp.zeros((GC, 5), jnp.float32)], axis=1)
        mm = lax.dot_general(ca, pb, (((1,), (0,)), ((), ())),
                             precision=lax.Precision.DEFAULT,
                             preferred_element_type=jnp.float32)
        d = (-2.0 * mm + c2) + x2

        lane = lax.broadcasted_iota(jnp.int32, (GC, NL), 1)
        L = jnp.full((GC, NL), -_INF, jnp.float32)
        V = []
        I = []
        NW = 4
        for _ in range(NR):
            ms = [jnp.full((GC, NL), _INF, jnp.float32) for _ in range(NW)]
            cis = [jnp.zeros((GC, NL), jnp.int32) for _ in range(NW)]
            for c in range(NCH):
                w = c % NW
                dc = d[:, c * NL:(c + 1) * NL]
                cand = jnp.where(dc > L, dc, _INF)
                lt = cand < ms[w]
                ms[w] = jnp.where(lt, cand, ms[w])
                cis[w] = jnp.where(lt, c, cis[w])
            m, ci = ms[0], cis[0]
            for w in range(1, NW):
                lt = ms[w] < m
                m = jnp.where(lt, ms[w], m)
                ci = jnp.where(lt, cis[w], ci)
            V.append(m)
            I.append(ci * NL + lane)
            L = m
        V.append(jnp.full((GC, NL), _INF, jnp.float32))
        I.append(jnp.full((GC, NL), _IBIG, jnp.int32))

        mcol = lax.broadcasted_iota(jnp.int32, (GC, NL), 1)
        acc = jnp.zeros((GC, NL), jnp.int32)
        for t in range(M):
            m0 = jnp.min(V[0], axis=1, keepdims=True)
            eq = V[0] == m0
            sel = jnp.min(jnp.where(eq, I[0], _IBIG), axis=1, keepdims=True)
            acc = jnp.where(mcol == t, sel, acc)
            win = eq & (I[0] == sel)
            for j in range(NR):
                V[j] = jnp.where(win, V[j + 1], V[j])
                I[j] = jnp.where(win, I[j + 1], I[j])
        out_ref[0, pl.ds(gc * GC, GC), :] = acc
        return 0

    lax.fori_loop(0, G // GC, chunk, 0)


def _knn_call(x, y, z, cx, cy, cz):
    cstack = jnp.stack([cx, cy, cz], axis=-1)
    cpad = jnp.concatenate(
        [cstack, jnp.zeros((B, G, NL - 3), jnp.float32)], axis=-1)
    plane = pl.BlockSpec((B, N), lambda i: (0, 0))
    out = pl.pallas_call(
        _knn_body,
        grid=(B,),
        in_specs=[plane, plane, plane,
                  pl.BlockSpec((1, G, NL), lambda i: (i, 0, 0))],
        out_specs=pl.BlockSpec((1, G, NL), lambda i: (i, 0, 0)),
        out_shape=jax.ShapeDtypeStruct((B, G, NL), jnp.int32),
    )(x, y, z, cpad)
    return out[:, :, :M]


def _gather_body(xp, yp, zp, cxp, cyp, czp, ip,
                 nx_out, ny_out, nz_out,
                 xv, yv, zv, cxv, cyv, czv, iv, ox, oy, oz):
    nc = 2
    b = lax.axis_index("s") * nc + lax.axis_index("c")
    pltpu.sync_copy(xp.at[b], xv)
    pltpu.sync_copy(yp.at[b], yv)
    pltpu.sync_copy(zp.at[b], zv)
    pltpu.sync_copy(cxp.at[b], cxv)
    pltpu.sync_copy(cyp.at[b], cyv)
    pltpu.sync_copy(czp.at[b], czv)
    pltpu.sync_copy(ip.at[b], iv)

    lane = lax.broadcasted_iota(jnp.int32, (16,), 0)

    def body(j, _):
        base = j * 16
        idx16 = iv[pl.ds(base, 16)]
        g16 = lax.shift_right_logical(base + lane, 5)
        px = plsc.load_gather(xv, [idx16])
        py = plsc.load_gather(yv, [idx16])
        pz = plsc.load_gather(zv, [idx16])
        cx16 = plsc.load_gather(cxv, [g16])
        cy16 = plsc.load_gather(cyv, [g16])
        cz16 = plsc.load_gather(czv, [g16])
        ox[pl.ds(base, 16)] = px - cx16
        oy[pl.ds(base, 16)] = py - cy16
        oz[pl.ds(base, 16)] = pz - cz16
        return 0

    lax.fori_loop(0, (G * M) // 16, body, 0)
    pltpu.sync_copy(ox, nx_out.at[b])
    pltpu.sync_copy(oy, ny_out.at[b])
    pltpu.sync_copy(oz, nz_out.at[b])


def _gather_call(xp, yp, zp, cx, cy, cz, idx):
    f32 = jnp.float32
    gm = G * M
    out_type = (
        jax.ShapeDtypeStruct((B, gm), f32),
        jax.ShapeDtypeStruct((B, gm), f32),
        jax.ShapeDtypeStruct((B, gm), f32),
    )
    scratch = [
        pltpu.VMEM((N,), f32), pltpu.VMEM((N,), f32), pltpu.VMEM((N,), f32),
        pltpu.VMEM((G,), f32), pltpu.VMEM((G,), f32), pltpu.VMEM((G,), f32),
        pltpu.VMEM((gm,), jnp.int32),
        pltpu.VMEM((gm,), f32), pltpu.VMEM((gm,), f32), pltpu.VMEM((gm,), f32),
    ]
    mesh = plsc.VectorSubcoreMesh(core_axis_name="c", subcore_axis_name="s")
    fn = pl.kernel(_gather_body, out_type=out_type, mesh=mesh,
                   scratch_types=scratch,
                   compiler_params=pltpu.CompilerParams(
                       needs_layout_passes=False))
    return fn(xp, yp, zp, cx, cy, cz, idx)


def kernel(xyz):
    x = xyz[:, :, 0]
    y = xyz[:, :, 1]
    z = xyz[:, :, 2]
    fps_idx, cx, cy, cz = _fps_call(x, y, z)
    gi = _knn_call(x, y, z, cx, cy, cz)
    idx = gi.reshape(B, G * M)
    nx, ny, nz = _gather_call(x, y, z, cx, cy, cz, idx)
    neighborhood = jnp.stack([nx, ny, nz], axis=-1).reshape(B, G, M, 3)
    center = jnp.stack([cx, cy, cz], axis=-1)
    return (neighborhood, center)
